# Initial kernel scaffold; baseline (speedup 1.0000x reference)
import jax
import jax.numpy as jnp
from jax.experimental import pallas as pl


def kernel(x, edge_index, W1, b1, W2, b2, W3, b3, Wn1, bn1, Wn2, bn2, Wg1, bg1, Wg2, bg2):
    raise NotImplementedError("write your pallas kernel here")


# speedup vs baseline: 2.2613x; 2.2613x over previous
import functools

import jax
import jax.numpy as jnp
from jax import lax
from jax.experimental import pallas as pl
from jax.experimental.pallas import tpu as pltpu


def _mlp_body(h_ref, wn1_ref, bn1_ref, wn2_ref, bn2_ref, out_ref):
    h = h_ref[...]
    t = jnp.maximum(h @ wn1_ref[...] + bn1_ref[...], 0.0)
    out_ref[...] = jnp.maximum(t @ wn2_ref[...] + bn2_ref[...], 0.0)


def _node_mlp(h, Wn1, bn1, Wn2, bn2):
    n = h.shape[0]
    blk = 2000
    assert n % blk == 0
    grid = (n // blk,)
    return pl.pallas_call(
        _mlp_body,
        grid=grid,
        in_specs=[
            pl.BlockSpec((blk, h.shape[1]), lambda i: (i, 0)),
            pl.BlockSpec(Wn1.shape, lambda i: (0, 0)),
            pl.BlockSpec((1, bn1.shape[0]), lambda i: (0, 0)),
            pl.BlockSpec(Wn2.shape, lambda i: (0, 0)),
            pl.BlockSpec((1, bn2.shape[0]), lambda i: (0, 0)),
        ],
        out_specs=pl.BlockSpec((blk, Wn2.shape[1]), lambda i: (i, 0)),
        out_shape=jax.ShapeDtypeStruct((n, Wn2.shape[1]), jnp.float32),
    )(h, Wn1, bn1.reshape(1, -1), Wn2, bn2.reshape(1, -1))


def kernel(x, edge_index, W1, b1, W2, b2, W3, b3, Wn1, bn1, Wn2, bn2, Wg1, bg1, Wg2, bg2):
    src = edge_index[0]
    dst = edge_index[1]
    n = x.shape[0]

    deg = jax.ops.segment_sum(jnp.ones(dst.shape[0], jnp.float32), dst,
                              num_segments=n) + 1.0
    dinv = lax.rsqrt(deg)

    def layer(h, W, b, act):
        u = dinv[:, None] * (h @ W)
        acc = jax.ops.segment_sum(u[src], dst, num_segments=n)
        out = dinv[:, None] * (acc + u) + b
        return jnp.maximum(out, 0.0) if act else out

    h = layer(x, W1, b1, True)
    h = layer(h, W2, b2, True)
    h = layer(h, W3, b3, False)

    node_features = _node_mlp(h, Wn1, bn1, Wn2, bn2)
    g = jnp.mean(h, axis=0, keepdims=True)
    global_features = jnp.maximum(
        jnp.maximum(g @ Wg1 + bg1, 0.0) @ Wg2 + bg2, 0.0)
    return (node_features, global_features)


# speedup vs baseline: 2.3957x; 1.0594x over previous
import functools

import jax
import jax.numpy as jnp
from jax import lax
from jax.experimental import pallas as pl
from jax.experimental.pallas import tpu as pltpu
from jax.experimental.pallas import tpu_sc as plsc

N = 100000
E = 1600000
NCHUNK = 8192
NUM_CHUNKS = 13
NP = NUM_CHUNKS * NCHUNK
ER = 12544
EPAD = ER * 128 - E
WROWS = 16
ROWS_PER_TILE = ER // 16
DEG_STRIPE = NP // 16
BLK = 2000


def _rsqrt_newton(v):
    xhalf = v * 0.5
    i = lax.bitcast_convert_type(v, jnp.int32)
    i = jnp.int32(0x5F3759DF) - lax.shift_right_arithmetic(i, 1)
    y = lax.bitcast_convert_type(i, jnp.float32)
    for _ in range(3):
        y = y * (1.5 - xhalf * y * y)
    return y


def _deg_body(dstp, dinv_out, dwin, ones_v, stripe_v, dtile, deg_sh, sem):
    cid = lax.axis_index("c")
    sid = lax.axis_index("s")

    def _zero16(i, _):
        stripe_v[pl.ds(i * 16, 16)] = jnp.zeros((16,), jnp.float32)
        return 0

    lax.fori_loop(0, DEG_STRIPE // 16, _zero16, 0)
    for i in range(8):
        ones_v[pl.ds(i * 16, 16)] = jnp.ones((16,), jnp.float32)
    pltpu.sync_copy(stripe_v, deg_sh.at[pl.ds(sid * DEG_STRIPE, DEG_STRIPE)])
    plsc.subcore_barrier()

    nwin = ROWS_PER_TILE // WROWS

    def _win(w, _):
        base_row = sid * ROWS_PER_TILE + w * WROWS
        pltpu.async_copy(dstp.at[pl.ds(base_row, WROWS)], dwin, sem).wait()

        def _row(r, _):
            pltpu.sync_copy(ones_v, deg_sh.at[dwin.at[r]], add=True)
            return 0

        lax.fori_loop(0, WROWS, _row, 0)
        return 0

    lax.fori_loop(0, nwin, _win, 0)
    plsc.subcore_barrier()

    half = NP // 2
    off = cid * half + sid * (half // 16)
    pltpu.sync_copy(deg_sh.at[pl.ds(off, half // 16)], dtile)

    def _rs(i, _):
        v = dtile[pl.ds(i * 16, 16)] + 1.0
        dtile[pl.ds(i * 16, 16)] = _rsqrt_newton(v)
        return 0

    lax.fori_loop(0, (half // 16) // 16, _rs, 0)
    pltpu.sync_copy(dtile, dinv_out.at[pl.ds(off, half // 16)])


def _deg_dinv(dstp):
    mesh = plsc.VectorSubcoreMesh(core_axis_name="c", subcore_axis_name="s")
    return pl.kernel(
        _deg_body,
        out_type=jax.ShapeDtypeStruct((NP,), jnp.float32),
        mesh=mesh,
        scratch_types=[
            pltpu.VMEM((WROWS, 128), jnp.int32),
            pltpu.VMEM((128,), jnp.float32),
            pltpu.VMEM((DEG_STRIPE,), jnp.float32),
            pltpu.VMEM((NP // 32,), jnp.float32),
            pltpu.VMEM_SHARED((NP,), jnp.float32),
            pltpu.SemaphoreType.DMA,
        ],
    )(dstp)


def _tc_a_body(x_ref, w_ref, dinv_ref, out_ref):
    out_ref[...] = dinv_ref[...] * (x_ref[...] @ w_ref[...])


def _tc_a(x, w, dinv2d):
    fo = w.shape[1]
    return pl.pallas_call(
        _tc_a_body,
        grid=(N // BLK,),
        in_specs=[
            pl.BlockSpec((BLK, x.shape[1]), lambda i: (i, 0)),
            pl.BlockSpec(w.shape, lambda i: (0, 0)),
            pl.BlockSpec((BLK, 1), lambda i: (i, 0)),
        ],
        out_specs=pl.BlockSpec((BLK, fo), lambda i: (i, 0)),
        out_shape=jax.ShapeDtypeStruct((N, fo), jnp.float32),
    )(x, w, dinv2d)


def _tc_b_body(acc_ref, u_ref, dinv_ref, b_ref, w_ref, out_ref):
    dv = dinv_ref[...]
    h = jnp.maximum(dv * (acc_ref[...] + u_ref[...]) + b_ref[...], 0.0)
    out_ref[...] = dv * (h @ w_ref[...])


def _tc_b(acc, u, dinv2d, b, w):
    f, fo = w.shape
    return pl.pallas_call(
        _tc_b_body,
        grid=(N // BLK,),
        in_specs=[
            pl.BlockSpec((BLK, f), lambda i: (i, 0)),
            pl.BlockSpec((BLK, f), lambda i: (i, 0)),
            pl.BlockSpec((BLK, 1), lambda i: (i, 0)),
            pl.BlockSpec((1, f), lambda i: (0, 0)),
            pl.BlockSpec(w.shape, lambda i: (0, 0)),
        ],
        out_specs=pl.BlockSpec((BLK, fo), lambda i: (i, 0)),
        out_shape=jax.ShapeDtypeStruct((N, fo), jnp.float32),
    )(acc, u, dinv2d, b.reshape(1, -1), w)


def _tc_c_body(acc_ref, u_ref, dinv_ref, b_ref, wn1_ref, bn1_ref, wn2_ref,
               bn2_ref, wg1_ref, bg1_ref, wg2_ref, bg2_ref, nf_ref, gf_ref,
               gsum):
    i = pl.program_id(0)
    dv = dinv_ref[...]
    h3 = dv * (acc_ref[...] + u_ref[...]) + b_ref[...]
    t = jnp.maximum(h3 @ wn1_ref[...] + bn1_ref[...], 0.0)
    nf_ref[...] = jnp.maximum(t @ wn2_ref[...] + bn2_ref[...], 0.0)
    s = jnp.sum(h3, axis=0, keepdims=True)
    prev = jnp.where(i == 0, jnp.zeros_like(s), gsum[...])
    tot = prev + s
    gsum[...] = tot

    @pl.when(i == (N // BLK) - 1)
    def _():
        g = tot * (1.0 / N)
        gg = jnp.maximum(g @ wg1_ref[...] + bg1_ref[...], 0.0)
        gf_ref[...] = jnp.maximum(gg @ wg2_ref[...] + bg2_ref[...], 0.0)


def _tc_c(acc, u, dinv2d, b3, Wn1, bn1, Wn2, bn2, Wg1, bg1, Wg2, bg2):
    full = lambda a: pl.BlockSpec(a.shape, lambda i: tuple(0 for _ in a.shape))
    row = lambda a: pl.BlockSpec((1, a.shape[0]), lambda i: (0, 0))
    return pl.pallas_call(
        _tc_c_body,
        grid=(N // BLK,),
        in_specs=[
            pl.BlockSpec((BLK, 64), lambda i: (i, 0)),
            pl.BlockSpec((BLK, 64), lambda i: (i, 0)),
            pl.BlockSpec((BLK, 1), lambda i: (i, 0)),
            row(b3), full(Wn1), row(bn1), full(Wn2), row(bn2),
            full(Wg1), row(bg1), full(Wg2), row(bg2),
        ],
        out_specs=[
            pl.BlockSpec((BLK, 64), lambda i: (i, 0)),
            pl.BlockSpec((1, 64), lambda i: (0, 0)),
        ],
        out_shape=[
            jax.ShapeDtypeStruct((N, 64), jnp.float32),
            jax.ShapeDtypeStruct((1, 64), jnp.float32),
        ],
        scratch_shapes=[pltpu.VMEM((1, 64), jnp.float32)],
    )(acc, u, dinv2d, b3.reshape(1, -1), Wn1, bn1.reshape(1, -1), Wn2,
      bn2.reshape(1, -1), Wg1, bg1.reshape(1, -1), Wg2, bg2.reshape(1, -1))


def kernel(x, edge_index, W1, b1, W2, b2, W3, b3, Wn1, bn1, Wn2, bn2, Wg1, bg1, Wg2, bg2):
    src = edge_index[0].astype(jnp.int32)
    dst = edge_index[1].astype(jnp.int32)

    padix = jnp.arange(EPAD, dtype=jnp.int32) % 128
    srcp = jnp.concatenate([src, padix]).reshape(ER, 128)
    dstp = jnp.concatenate([dst, NP - 128 + padix]).reshape(ER, 128)

    dinv1 = _deg_dinv(dstp)
    dinv2d = dinv1[:N, None]

    def segsum(u):
        return jax.ops.segment_sum(u[src], dst, num_segments=N)

    u1 = _tc_a(x, W1, dinv2d)
    acc1 = segsum(u1)
    u2 = _tc_b(acc1, u1, dinv2d, b1, W2)
    acc2 = segsum(u2)
    u3 = _tc_b(acc2, u2, dinv2d, b2, W3)
    acc3 = segsum(u3)
    nf, gf = _tc_c(acc3, u3, dinv2d, b3, Wn1, bn1, Wn2, bn2, Wg1, bg1,
                   Wg2, bg2)
    return (nf, gf)


# speedup vs baseline: 5.9651x; 2.4899x over previous
import functools

import jax
import jax.numpy as jnp
from jax import lax
from jax.experimental import pallas as pl
from jax.experimental.pallas import tpu as pltpu
from jax.experimental.pallas import tpu_sc as plsc

N = 100000
E = 1600000
NCHUNK = 8192
NUM_CHUNKS = 13
NP = NUM_CHUNKS * NCHUNK
ER = 12544
EPAD = ER * 128 - E
WROWS = 16
ROWS_PER_TILE = ER // 16
DEG_STRIPE = NP // 16
BLK = 2000


def _rsqrt_newton(v):
    xhalf = v * 0.5
    i = lax.bitcast_convert_type(v, jnp.int32)
    i = jnp.int32(0x5F3759DF) - lax.shift_right_arithmetic(i, 1)
    y = lax.bitcast_convert_type(i, jnp.float32)
    for _ in range(3):
        y = y * (1.5 - xhalf * y * y)
    return y


def _deg_body(dstp, dinv_out, dwin, ones_v, stripe_v, dtile, deg_sh, sem):
    cid = lax.axis_index("c")
    sid = lax.axis_index("s")

    def _zero16(i, _):
        stripe_v[pl.ds(i * 16, 16)] = jnp.zeros((16,), jnp.float32)
        return 0

    lax.fori_loop(0, DEG_STRIPE // 16, _zero16, 0)
    for i in range(8):
        ones_v[pl.ds(i * 16, 16)] = jnp.ones((16,), jnp.float32)
    pltpu.sync_copy(stripe_v, deg_sh.at[pl.ds(sid * DEG_STRIPE, DEG_STRIPE)])
    plsc.subcore_barrier()

    nwin = ROWS_PER_TILE // WROWS

    def _win(w, _):
        base_row = sid * ROWS_PER_TILE + w * WROWS
        pltpu.async_copy(dstp.at[pl.ds(base_row, WROWS)], dwin, sem).wait()

        def _row(r, _):
            pltpu.sync_copy(ones_v, deg_sh.at[dwin.at[r]], add=True)
            return 0

        lax.fori_loop(0, WROWS, _row, 0)
        return 0

    lax.fori_loop(0, nwin, _win, 0)
    plsc.subcore_barrier()

    half = NP // 2
    off = cid * half + sid * (half // 16)
    pltpu.sync_copy(deg_sh.at[pl.ds(off, half // 16)], dtile)

    def _rs(i, _):
        v = dtile[pl.ds(i * 16, 16)] + 1.0
        dtile[pl.ds(i * 16, 16)] = _rsqrt_newton(v)
        return 0

    lax.fori_loop(0, (half // 16) // 16, _rs, 0)
    pltpu.sync_copy(dtile, dinv_out.at[pl.ds(off, half // 16)])


def _deg_dinv(dstp):
    mesh = plsc.VectorSubcoreMesh(core_axis_name="c", subcore_axis_name="s")
    return pl.kernel(
        _deg_body,
        out_type=jax.ShapeDtypeStruct((NP,), jnp.float32),
        mesh=mesh,
        scratch_types=[
            pltpu.VMEM((WROWS, 128), jnp.int32),
            pltpu.VMEM((128,), jnp.float32),
            pltpu.VMEM((DEG_STRIPE,), jnp.float32),
            pltpu.VMEM((NP // 32,), jnp.float32),
            pltpu.VMEM_SHARED((NP,), jnp.float32),
            pltpu.SemaphoreType.DMA,
        ],
    )(dstp)


FSLOTS = 32
FLUSH = FSLOTS * 16


def _agg_body(f, nchunk, nchunks, u_hbm, srcp, dstp, acc_out, swin, dwin,
              pend, gsrc, gdst, rows, zbuf, acc_sh, sem, gsem):
    cid = lax.axis_index("c")
    sid = lax.axis_index("s")
    psh = 14 if nchunk == 8192 else 15
    lmask = (1 << psh) - 1
    acc_rows = nchunk + 128
    acc_stripe = acc_rows // 16
    wb_stripe = nchunk // 16
    lane = lax.iota(jnp.int32, 16)

    def _z16(i, _):
        r = i // (f // 16)
        o = (i % (f // 16)) * 16
        zbuf[r, pl.ds(o, 16)] = jnp.zeros((16,), jnp.float32)
        return 0

    lax.fori_loop(0, 128 * (f // 16), _z16, 0)

    def _pad_and_flush(cntl):
        for k in range(FSLOTS):
            padv = lax.shift_left(lane + 16 * k, psh) | (nchunk + lane)
            plsc.store_scatter(pend, [jnp.full((16,), k * 16, jnp.int32) + lane],
                               padv, mask=cntl <= k)

        def _up(k, _):
            pv = pend[pl.ds(k * 16, 16)]
            r = k // 8
            o = (k % 8) * 16
            gsrc[r, pl.ds(o, 16)] = lax.shift_right_logical(pv, psh)
            gdst[r, pl.ds(o, 16)] = pv & lmask
            return 0

        lax.fori_loop(0, FLUSH // 16, _up, 0)
        cps = [pltpu.async_copy(u_hbm.at[gsrc.at[j]],
                                rows.at[pl.ds(j * 128, 128)], gsem)
               for j in range(4)]
        for cp in cps:
            cp.wait()
        for j in range(4):
            pltpu.sync_copy(rows.at[pl.ds(j * 128, 128)],
                            acc_sh.at[gdst.at[j]], add=True)
        return jnp.zeros((16,), jnp.int32)

    def _chunk(ci, _):
        base = (2 * ci + cid) * nchunk
        for k in range(4):
            pltpu.sync_copy(zbuf, acc_sh.at[pl.ds(sid * acc_stripe + k * 128,
                                                  128)])
        pltpu.sync_copy(zbuf.at[pl.ds(0, 8)],
                        acc_sh.at[pl.ds(sid * acc_stripe + 512, 8)])
        plsc.subcore_barrier()

        def _win(w, cntl):
            base_row = sid * ROWS_PER_TILE + w * WROWS
            c1 = pltpu.async_copy(srcp.at[pl.ds(base_row, WROWS)], swin, sem)
            c2 = pltpu.async_copy(dstp.at[pl.ds(base_row, WROWS)], dwin, sem)
            c1.wait()
            c2.wait()

            def _grp(g, cntl):
                r = g // 8
                o = (g % 8) * 16
                s16 = swin[r, pl.ds(o, 16)]
                d16 = dwin[r, pl.ds(o, 16)]
                dl = d16 - base
                m = (d16 >= base) & (dl < nchunk)
                v = lax.shift_left(s16, psh) | (dl & lmask)
                pos = lax.shift_left(cntl, 4) + lane
                plsc.store_scatter(pend, [pos], v, mask=m)
                cntl = cntl + jnp.where(m, 1, 0)
                return lax.cond(jnp.any(cntl >= FSLOTS), _pad_and_flush,
                                lambda c: c, cntl)

            return lax.fori_loop(0, WROWS * 8, _grp, cntl)

        cntl = lax.fori_loop(0, ROWS_PER_TILE // WROWS, _win,
                             jnp.zeros((16,), jnp.int32))
        cntl = lax.cond(jnp.any(cntl > 0), _pad_and_flush, lambda c: c, cntl)
        plsc.subcore_barrier()
        pltpu.sync_copy(acc_sh.at[pl.ds(sid * wb_stripe, wb_stripe)],
                        acc_out.at[pl.ds(base + sid * wb_stripe, wb_stripe)])
        plsc.subcore_barrier()
        return 0

    nch = jnp.where(cid == 0, (nchunks + 1) // 2, nchunks // 2)
    lax.fori_loop(0, nch, _chunk, 0)


def _sc_agg(u, srcp, dstp, nchunk):
    f = u.shape[1]
    nchunks = -(-NP // nchunk)
    mesh = plsc.VectorSubcoreMesh(core_axis_name="c", subcore_axis_name="s")
    return pl.kernel(
        functools.partial(_agg_body, f, nchunk, nchunks),
        out_type=jax.ShapeDtypeStruct((nchunks * nchunk, f), jnp.float32),
        mesh=mesh,
        compiler_params=pltpu.CompilerParams(needs_layout_passes=False, use_tc_tiling_on_sc=False),
        scratch_types=[
            pltpu.VMEM((WROWS, 128), jnp.int32),
            pltpu.VMEM((WROWS, 128), jnp.int32),
            pltpu.VMEM((FLUSH,), jnp.int32),
            pltpu.VMEM((4, 128), jnp.int32),
            pltpu.VMEM((4, 128), jnp.int32),
            pltpu.VMEM((FLUSH, f), jnp.float32),
            pltpu.VMEM((128, f), jnp.float32),
            pltpu.VMEM_SHARED((nchunk + 128, f), jnp.float32),
            pltpu.SemaphoreType.DMA,
            pltpu.SemaphoreType.DMA,
        ],
    )(u, srcp, dstp)


def _tc_a_body(x_ref, w_ref, dinv_ref, out_ref):
    out_ref[...] = dinv_ref[...] * (x_ref[...] @ w_ref[...])


def _tc_a(x, w, dinv2d):
    fo = w.shape[1]
    return pl.pallas_call(
        _tc_a_body,
        grid=(N // BLK,),
        in_specs=[
            pl.BlockSpec((BLK, x.shape[1]), lambda i: (i, 0)),
            pl.BlockSpec(w.shape, lambda i: (0, 0)),
            pl.BlockSpec((BLK, 1), lambda i: (i, 0)),
        ],
        out_specs=pl.BlockSpec((BLK, fo), lambda i: (i, 0)),
        out_shape=jax.ShapeDtypeStruct((N, fo), jnp.float32),
    )(x, w, dinv2d)


def _tc_b_body(acc_ref, u_ref, dinv_ref, b_ref, w_ref, out_ref):
    dv = dinv_ref[...]
    h = jnp.maximum(dv * (acc_ref[...] + u_ref[...]) + b_ref[...], 0.0)
    out_ref[...] = dv * (h @ w_ref[...])


def _tc_b(acc, u, dinv2d, b, w):
    f, fo = w.shape
    return pl.pallas_call(
        _tc_b_body,
        grid=(N // BLK,),
        in_specs=[
            pl.BlockSpec((BLK, f), lambda i: (i, 0)),
            pl.BlockSpec((BLK, f), lambda i: (i, 0)),
            pl.BlockSpec((BLK, 1), lambda i: (i, 0)),
            pl.BlockSpec((1, f), lambda i: (0, 0)),
            pl.BlockSpec(w.shape, lambda i: (0, 0)),
        ],
        out_specs=pl.BlockSpec((BLK, fo), lambda i: (i, 0)),
        out_shape=jax.ShapeDtypeStruct((N, fo), jnp.float32),
    )(acc, u, dinv2d, b.reshape(1, -1), w)


def _tc_c_body(acc_ref, u_ref, dinv_ref, b_ref, wn1_ref, bn1_ref, wn2_ref,
               bn2_ref, wg1_ref, bg1_ref, wg2_ref, bg2_ref, nf_ref, gf_ref,
               gsum):
    i = pl.program_id(0)
    dv = dinv_ref[...]
    h3 = dv * (acc_ref[...] + u_ref[...]) + b_ref[...]
    t = jnp.maximum(h3 @ wn1_ref[...] + bn1_ref[...], 0.0)
    nf_ref[...] = jnp.maximum(t @ wn2_ref[...] + bn2_ref[...], 0.0)
    s = jnp.sum(h3, axis=0, keepdims=True)
    prev = jnp.where(i == 0, jnp.zeros_like(s), gsum[...])
    tot = prev + s
    gsum[...] = tot

    @pl.when(i == (N // BLK) - 1)
    def _():
        g = tot * (1.0 / N)
        gg = jnp.maximum(g @ wg1_ref[...] + bg1_ref[...], 0.0)
        gf_ref[...] = jnp.maximum(gg @ wg2_ref[...] + bg2_ref[...], 0.0)


def _tc_c(acc, u, dinv2d, b3, Wn1, bn1, Wn2, bn2, Wg1, bg1, Wg2, bg2):
    full = lambda a: pl.BlockSpec(a.shape, lambda i: tuple(0 for _ in a.shape))
    row = lambda a: pl.BlockSpec((1, a.shape[0]), lambda i: (0, 0))
    return pl.pallas_call(
        _tc_c_body,
        grid=(N // BLK,),
        in_specs=[
            pl.BlockSpec((BLK, 64), lambda i: (i, 0)),
            pl.BlockSpec((BLK, 64), lambda i: (i, 0)),
            pl.BlockSpec((BLK, 1), lambda i: (i, 0)),
            row(b3), full(Wn1), row(bn1), full(Wn2), row(bn2),
            full(Wg1), row(bg1), full(Wg2), row(bg2),
        ],
        out_specs=[
            pl.BlockSpec((BLK, 64), lambda i: (i, 0)),
            pl.BlockSpec((1, 64), lambda i: (0, 0)),
        ],
        out_shape=[
            jax.ShapeDtypeStruct((N, 64), jnp.float32),
            jax.ShapeDtypeStruct((1, 64), jnp.float32),
        ],
        scratch_shapes=[pltpu.VMEM((1, 64), jnp.float32)],
    )(acc, u, dinv2d, b3.reshape(1, -1), Wn1, bn1.reshape(1, -1), Wn2,
      bn2.reshape(1, -1), Wg1, bg1.reshape(1, -1), Wg2, bg2.reshape(1, -1))


def kernel(x, edge_index, W1, b1, W2, b2, W3, b3, Wn1, bn1, Wn2, bn2, Wg1, bg1, Wg2, bg2):
    src = edge_index[0].astype(jnp.int32)
    dst = edge_index[1].astype(jnp.int32)

    padix = jnp.arange(EPAD, dtype=jnp.int32) % 128
    srcp = jnp.concatenate([src, padix]).reshape(ER, 128)
    dstp = jnp.concatenate([dst, NP - 128 + padix]).reshape(ER, 128)

    dinv1 = _deg_dinv(dstp)
    dinv2d = dinv1[:N, None]

    u1 = _tc_a(x, W1, dinv2d)
    acc1 = _sc_agg(u1, srcp, dstp, 8192)
    u2 = _tc_b(acc1, u1, dinv2d, b1, W2)
    acc2 = _sc_agg(u2, srcp, dstp, 4096)
    u3 = _tc_b(acc2, u2, dinv2d, b2, W3)
    acc3 = _sc_agg(u3, srcp, dstp, 8192)
    nf, gf = _tc_c(acc3, u3, dinv2d, b3, Wn1, bn1, Wn2, bn2, Wg1, bg1,
                   Wg2, bg2)
    return (nf, gf)


# speedup vs baseline: 8.0206x; 1.3446x over previous
import functools

import jax
import jax.numpy as jnp
from jax import lax
from jax.experimental import pallas as pl
from jax.experimental.pallas import tpu as pltpu
from jax.experimental.pallas import tpu_sc as plsc

N = 100000
E = 1600000
NCHUNK = 8192
NUM_CHUNKS = 13
NP = NUM_CHUNKS * NCHUNK
ER = 12544
EPAD = ER * 128 - E
WROWS = 16
ROWS_PER_TILE = ER // 16
DEG_STRIPE = NP // 16
BLK = 2000


def _rsqrt_newton(v):
    xhalf = v * 0.5
    i = lax.bitcast_convert_type(v, jnp.int32)
    i = jnp.int32(0x5F3759DF) - lax.shift_right_arithmetic(i, 1)
    y = lax.bitcast_convert_type(i, jnp.float32)
    for _ in range(3):
        y = y * (1.5 - xhalf * y * y)
    return y


def _deg_body(dstp, dinv_out, dwin, ones_v, stripe_v, dtile, deg_sh, sem):
    cid = lax.axis_index("c")
    sid = lax.axis_index("s")

    def _zero16(i, _):
        stripe_v[pl.ds(i * 16, 16)] = jnp.zeros((16,), jnp.float32)
        return 0

    lax.fori_loop(0, DEG_STRIPE // 16, _zero16, 0)
    for i in range(8):
        ones_v[pl.ds(i * 16, 16)] = jnp.ones((16,), jnp.float32)
    pltpu.sync_copy(stripe_v, deg_sh.at[pl.ds(sid * DEG_STRIPE, DEG_STRIPE)])
    plsc.subcore_barrier()

    nwin = ROWS_PER_TILE // WROWS

    def _win(w, _):
        base_row = sid * ROWS_PER_TILE + w * WROWS
        pltpu.async_copy(dstp.at[pl.ds(base_row, WROWS)], dwin, sem).wait()

        def _row(r, _):
            pltpu.sync_copy(ones_v, deg_sh.at[dwin.at[r]], add=True)
            return 0

        lax.fori_loop(0, WROWS, _row, 0)
        return 0

    lax.fori_loop(0, nwin, _win, 0)
    plsc.subcore_barrier()

    half = NP // 2
    off = cid * half + sid * (half // 16)
    pltpu.sync_copy(deg_sh.at[pl.ds(off, half // 16)], dtile)

    def _rs(i, _):
        v = dtile[pl.ds(i * 16, 16)] + 1.0
        dtile[pl.ds(i * 16, 16)] = _rsqrt_newton(v)
        return 0

    lax.fori_loop(0, (half // 16) // 16, _rs, 0)
    pltpu.sync_copy(dtile, dinv_out.at[pl.ds(off, half // 16)])


def _deg_dinv(dstp):
    mesh = plsc.VectorSubcoreMesh(core_axis_name="c", subcore_axis_name="s")
    return pl.kernel(
        _deg_body,
        out_type=jax.ShapeDtypeStruct((NP,), jnp.float32),
        mesh=mesh,
        scratch_types=[
            pltpu.VMEM((WROWS, 128), jnp.int32),
            pltpu.VMEM((128,), jnp.float32),
            pltpu.VMEM((DEG_STRIPE,), jnp.float32),
            pltpu.VMEM((NP // 32,), jnp.float32),
            pltpu.VMEM_SHARED((NP,), jnp.float32),
            pltpu.SemaphoreType.DMA,
        ],
    )(dstp)


FSLOTS = 32
FLUSH = FSLOTS * 16


def _agg_body(f, nchunk, nchunks, u_hbm, srcp, dstp, acc_out, swin, dwin,
              pend, gsrc, gdst, rows, zbuf, acc_sh, sem, gsem):
    cid = lax.axis_index("c")
    sid = lax.axis_index("s")
    psh = 14 if nchunk == 8192 else 15
    lmask = (1 << psh) - 1
    acc_rows = nchunk + 128
    acc_stripe = acc_rows // 16
    wb_stripe = nchunk // 16
    lane = lax.iota(jnp.int32, 16)

    def _z16(i, _):
        r = i // (f // 16)
        o = (i % (f // 16)) * 16
        zbuf[r, pl.ds(o, 16)] = jnp.zeros((16,), jnp.float32)
        return 0

    lax.fori_loop(0, 128 * (f // 16), _z16, 0)

    def _pad_and_flush(cntl):
        for k in range(FSLOTS):
            padv = lax.shift_left(lane + 16 * k, psh) | (nchunk + lane)
            plsc.store_scatter(pend, [jnp.full((16,), k * 16, jnp.int32) + lane],
                               padv, mask=cntl <= k)

        def _up(k, _):
            pv = pend[pl.ds(k * 16, 16)]
            r = k // 8
            o = (k % 8) * 16
            gsrc[r, pl.ds(o, 16)] = lax.shift_right_logical(pv, psh)
            gdst[r, pl.ds(o, 16)] = pv & lmask
            return 0

        lax.fori_loop(0, FLUSH // 16, _up, 0)
        cps = [pltpu.async_copy(u_hbm.at[gsrc.at[j]],
                                rows.at[pl.ds(j * 128, 128)], gsem)
               for j in range(4)]
        for cp in cps:
            cp.wait()
        for j in range(4):
            pltpu.sync_copy(rows.at[pl.ds(j * 128, 128)],
                            acc_sh.at[gdst.at[j]], add=True)
        return jnp.zeros((16,), jnp.int32)

    def _chunk(ci, _):
        base = (2 * ci + cid) * nchunk
        for k in range(4):
            pltpu.sync_copy(zbuf, acc_sh.at[pl.ds(sid * acc_stripe + k * 128,
                                                  128)])
        pltpu.sync_copy(zbuf.at[pl.ds(0, 8)],
                        acc_sh.at[pl.ds(sid * acc_stripe + 512, 8)])
        plsc.subcore_barrier()

        def _win(w, cntl):
            base_row = sid * ROWS_PER_TILE + w * WROWS
            c1 = pltpu.async_copy(srcp.at[pl.ds(base_row, WROWS)], swin, sem)
            c2 = pltpu.async_copy(dstp.at[pl.ds(base_row, WROWS)], dwin, sem)
            c1.wait()
            c2.wait()

            def _row(r, cntl):
                for g in range(8):
                    o = g * 16
                    s16 = swin[r, pl.ds(o, 16)]
                    d16 = dwin[r, pl.ds(o, 16)]
                    dl = d16 - base
                    m = (d16 >= base) & (dl < nchunk)
                    v = lax.shift_left(s16, psh) | (dl & lmask)
                    pos = lax.shift_left(cntl, 4) + lane
                    plsc.store_scatter(pend, [pos], v, mask=m)
                    cntl = cntl + jnp.where(m, 1, 0)
                return lax.cond(jnp.any(cntl >= FSLOTS - 8), _pad_and_flush,
                                lambda c: c, cntl)

            return lax.fori_loop(0, WROWS, _row, cntl)

        cntl = lax.fori_loop(0, ROWS_PER_TILE // WROWS, _win,
                             jnp.zeros((16,), jnp.int32))
        cntl = lax.cond(jnp.any(cntl > 0), _pad_and_flush, lambda c: c, cntl)
        plsc.subcore_barrier()
        pltpu.sync_copy(acc_sh.at[pl.ds(sid * wb_stripe, wb_stripe)],
                        acc_out.at[pl.ds(base + sid * wb_stripe, wb_stripe)])
        plsc.subcore_barrier()
        return 0

    nch = jnp.where(cid == 0, (nchunks + 1) // 2, nchunks // 2)
    lax.fori_loop(0, nch, _chunk, 0)


def _sc_agg(u, srcp, dstp, nchunk):
    f = u.shape[1]
    nchunks = -(-NP // nchunk)
    mesh = plsc.VectorSubcoreMesh(core_axis_name="c", subcore_axis_name="s")
    return pl.kernel(
        functools.partial(_agg_body, f, nchunk, nchunks),
        out_type=jax.ShapeDtypeStruct((nchunks * nchunk, f), jnp.float32),
        mesh=mesh,
        compiler_params=pltpu.CompilerParams(needs_layout_passes=False, use_tc_tiling_on_sc=False),
        scratch_types=[
            pltpu.VMEM((WROWS, 128), jnp.int32),
            pltpu.VMEM((WROWS, 128), jnp.int32),
            pltpu.VMEM((FLUSH,), jnp.int32),
            pltpu.VMEM((4, 128), jnp.int32),
            pltpu.VMEM((4, 128), jnp.int32),
            pltpu.VMEM((FLUSH, f), jnp.float32),
            pltpu.VMEM((128, f), jnp.float32),
            pltpu.VMEM_SHARED((nchunk + 128, f), jnp.float32),
            pltpu.SemaphoreType.DMA,
            pltpu.SemaphoreType.DMA,
        ],
    )(u, srcp, dstp)


def _tc_a_body(x_ref, w_ref, dinv_ref, out_ref):
    out_ref[...] = dinv_ref[...] * (x_ref[...] @ w_ref[...])


def _tc_a(x, w, dinv2d):
    fo = w.shape[1]
    return pl.pallas_call(
        _tc_a_body,
        grid=(N // BLK,),
        in_specs=[
            pl.BlockSpec((BLK, x.shape[1]), lambda i: (i, 0)),
            pl.BlockSpec(w.shape, lambda i: (0, 0)),
            pl.BlockSpec((BLK, 1), lambda i: (i, 0)),
        ],
        out_specs=pl.BlockSpec((BLK, fo), lambda i: (i, 0)),
        out_shape=jax.ShapeDtypeStruct((N, fo), jnp.float32),
    )(x, w, dinv2d)


def _tc_b_body(acc_ref, u_ref, dinv_ref, b_ref, w_ref, out_ref):
    dv = dinv_ref[...]
    h = jnp.maximum(dv * (acc_ref[...] + u_ref[...]) + b_ref[...], 0.0)
    out_ref[...] = dv * (h @ w_ref[...])


def _tc_b(acc, u, dinv2d, b, w):
    f, fo = w.shape
    return pl.pallas_call(
        _tc_b_body,
        grid=(N // BLK,),
        in_specs=[
            pl.BlockSpec((BLK, f), lambda i: (i, 0)),
            pl.BlockSpec((BLK, f), lambda i: (i, 0)),
            pl.BlockSpec((BLK, 1), lambda i: (i, 0)),
            pl.BlockSpec((1, f), lambda i: (0, 0)),
            pl.BlockSpec(w.shape, lambda i: (0, 0)),
        ],
        out_specs=pl.BlockSpec((BLK, fo), lambda i: (i, 0)),
        out_shape=jax.ShapeDtypeStruct((N, fo), jnp.float32),
    )(acc, u, dinv2d, b.reshape(1, -1), w)


def _tc_c_body(acc_ref, u_ref, dinv_ref, b_ref, wn1_ref, bn1_ref, wn2_ref,
               bn2_ref, wg1_ref, bg1_ref, wg2_ref, bg2_ref, nf_ref, gf_ref,
               gsum):
    i = pl.program_id(0)
    dv = dinv_ref[...]
    h3 = dv * (acc_ref[...] + u_ref[...]) + b_ref[...]
    t = jnp.maximum(h3 @ wn1_ref[...] + bn1_ref[...], 0.0)
    nf_ref[...] = jnp.maximum(t @ wn2_ref[...] + bn2_ref[...], 0.0)
    s = jnp.sum(h3, axis=0, keepdims=True)
    prev = jnp.where(i == 0, jnp.zeros_like(s), gsum[...])
    tot = prev + s
    gsum[...] = tot

    @pl.when(i == (N // BLK) - 1)
    def _():
        g = tot * (1.0 / N)
        gg = jnp.maximum(g @ wg1_ref[...] + bg1_ref[...], 0.0)
        gf_ref[...] = jnp.maximum(gg @ wg2_ref[...] + bg2_ref[...], 0.0)


def _tc_c(acc, u, dinv2d, b3, Wn1, bn1, Wn2, bn2, Wg1, bg1, Wg2, bg2):
    full = lambda a: pl.BlockSpec(a.shape, lambda i: tuple(0 for _ in a.shape))
    row = lambda a: pl.BlockSpec((1, a.shape[0]), lambda i: (0, 0))
    return pl.pallas_call(
        _tc_c_body,
        grid=(N // BLK,),
        in_specs=[
            pl.BlockSpec((BLK, 64), lambda i: (i, 0)),
            pl.BlockSpec((BLK, 64), lambda i: (i, 0)),
            pl.BlockSpec((BLK, 1), lambda i: (i, 0)),
            row(b3), full(Wn1), row(bn1), full(Wn2), row(bn2),
            full(Wg1), row(bg1), full(Wg2), row(bg2),
        ],
        out_specs=[
            pl.BlockSpec((BLK, 64), lambda i: (i, 0)),
            pl.BlockSpec((1, 64), lambda i: (0, 0)),
        ],
        out_shape=[
            jax.ShapeDtypeStruct((N, 64), jnp.float32),
            jax.ShapeDtypeStruct((1, 64), jnp.float32),
        ],
        scratch_shapes=[pltpu.VMEM((1, 64), jnp.float32)],
    )(acc, u, dinv2d, b3.reshape(1, -1), Wn1, bn1.reshape(1, -1), Wn2,
      bn2.reshape(1, -1), Wg1, bg1.reshape(1, -1), Wg2, bg2.reshape(1, -1))


def kernel(x, edge_index, W1, b1, W2, b2, W3, b3, Wn1, bn1, Wn2, bn2, Wg1, bg1, Wg2, bg2):
    src = edge_index[0].astype(jnp.int32)
    dst = edge_index[1].astype(jnp.int32)

    padix = jnp.arange(EPAD, dtype=jnp.int32) % 128
    srcp = jnp.concatenate([src, padix]).reshape(ER, 128)
    dstp = jnp.concatenate([dst, NP - 128 + padix]).reshape(ER, 128)

    dinv1 = _deg_dinv(dstp)
    dinv2d = dinv1[:N, None]

    u1 = _tc_a(x, W1, dinv2d)
    acc1 = _sc_agg(u1, srcp, dstp, 8192)
    u2 = _tc_b(acc1, u1, dinv2d, b1, W2)
    acc2 = _sc_agg(u2, srcp, dstp, 4096)
    u3 = _tc_b(acc2, u2, dinv2d, b2, W3)
    acc3 = _sc_agg(u3, srcp, dstp, 8192)
    nf, gf = _tc_c(acc3, u3, dinv2d, b3, Wn1, bn1, Wn2, bn2, Wg1, bg1,
                   Wg2, bg2)
    return (nf, gf)


# speedup vs baseline: 9.9626x; 1.2421x over previous
import functools

import jax
import jax.numpy as jnp
from jax import lax
from jax.experimental import pallas as pl
from jax.experimental.pallas import tpu as pltpu
from jax.experimental.pallas import tpu_sc as plsc

N = 100000
E = 1600000
NP = 106496
NU = 102000
ER = 12544
EPAD = ER * 128 - E
WROWS = 16
ROWS_PER_TILE = ER // 16
ROWS_PER_GTILE = ER // 32
DEG_STRIPE = NP // 16
BLK = 2000
NBK = 26
CPT = NBK * 16
EB = ER * 128 + 2 * 16384
DUMP = EB - 512
PSH = 14


def _rsqrt_newton(v):
    xhalf = v * 0.5
    i = lax.bitcast_convert_type(v, jnp.int32)
    i = jnp.int32(0x5F3759DF) - lax.shift_right_arithmetic(i, 1)
    y = lax.bitcast_convert_type(i, jnp.float32)
    for _ in range(3):
        y = y * (1.5 - xhalf * y * y)
    return y


def _deg_body(dstp, dinv_out, dwin, ones_v, stripe_v, dtile, deg_sh, sem):
    cid = lax.axis_index("c")
    sid = lax.axis_index("s")

    def _zero16(i, _):
        stripe_v[pl.ds(i * 16, 16)] = jnp.zeros((16,), jnp.float32)
        return 0

    lax.fori_loop(0, DEG_STRIPE // 16, _zero16, 0)
    for i in range(8):
        ones_v[pl.ds(i * 16, 16)] = jnp.ones((16,), jnp.float32)
    pltpu.sync_copy(stripe_v, deg_sh.at[pl.ds(sid * DEG_STRIPE, DEG_STRIPE)])
    plsc.subcore_barrier()

    def _win(w, _):
        base_row = sid * ROWS_PER_TILE + w * WROWS
        pltpu.async_copy(dstp.at[pl.ds(base_row, WROWS)], dwin, sem).wait()

        def _row(r, _):
            pltpu.sync_copy(ones_v, deg_sh.at[dwin.at[r]], add=True)
            return 0

        lax.fori_loop(0, WROWS, _row, 0)
        return 0

    lax.fori_loop(0, ROWS_PER_TILE // WROWS, _win, 0)
    plsc.subcore_barrier()

    half = NP // 2
    off = cid * half + sid * (half // 16)
    pltpu.sync_copy(deg_sh.at[pl.ds(off, half // 16)], dtile)

    def _rs(i, _):
        v = dtile[pl.ds(i * 16, 16)] + 1.0
        dtile[pl.ds(i * 16, 16)] = _rsqrt_newton(v)
        return 0

    lax.fori_loop(0, (half // 16) // 16, _rs, 0)
    pltpu.sync_copy(dtile, dinv_out.at[pl.ds(off, half // 16)])


def _deg_dinv(dstp):
    mesh = plsc.VectorSubcoreMesh(core_axis_name="c", subcore_axis_name="s")
    return pl.kernel(
        _deg_body,
        out_type=jax.ShapeDtypeStruct((NP,), jnp.float32),
        mesh=mesh,
        scratch_types=[
            pltpu.VMEM((WROWS, 128), jnp.int32),
            pltpu.VMEM((128,), jnp.float32),
            pltpu.VMEM((DEG_STRIPE,), jnp.float32),
            pltpu.VMEM((NP // 32,), jnp.float32),
            pltpu.VMEM_SHARED((NP,), jnp.float32),
            pltpu.SemaphoreType.DMA,
        ],
    )(dstp)


def _b1_body(dstp, cnt_out, dwin, cnt, sem):
    cid = lax.axis_index("c")
    sid = lax.axis_index("s")
    tid = cid * 16 + sid
    lane = lax.iota(jnp.int32, 16)

    for i in range(CPT // 16):
        cnt[pl.ds(i * 16, 16)] = jnp.zeros((16,), jnp.int32)

    def _win(w, _):
        base_row = tid * ROWS_PER_GTILE + w * 8
        pltpu.async_copy(dstp.at[pl.ds(base_row, 8)], dwin, sem).wait()

        def _row(r, _):
            for g in range(8):
                d16 = dwin[r, pl.ds(g * 16, 16)]
                cidx = lax.shift_left(lax.shift_right_logical(d16, 12),
                                      4) + lane
                c = plsc.load_gather(cnt, [cidx])
                plsc.store_scatter(cnt, [cidx], c + 1)
            return 0

        lax.fori_loop(0, 8, _row, 0)
        return 0

    lax.fori_loop(0, ROWS_PER_GTILE // 8, _win, 0)
    pltpu.sync_copy(cnt, cnt_out.at[pl.ds(pl.multiple_of(tid * CPT, 8), CPT)])


def _bucket1(dstp):
    mesh = plsc.VectorSubcoreMesh(core_axis_name="c", subcore_axis_name="s")
    return pl.kernel(
        _b1_body,
        out_type=jax.ShapeDtypeStruct((32 * CPT,), jnp.int32),
        mesh=mesh,
        compiler_params=pltpu.CompilerParams(needs_layout_passes=False,
                                             use_tc_tiling_on_sc=False),
        scratch_types=[
            pltpu.VMEM((8, 128), jnp.int32),
            pltpu.VMEM((CPT,), jnp.int32),
            pltpu.SemaphoreType.DMA,
        ],
    )(dstp)


def _excl_scan16(x, lane):
    inc = x
    for sh in (1, 2, 4, 8):
        g = inc.at[jnp.maximum(lane - sh, 0)].get(mode="promise_in_bounds")
        inc = inc + jnp.where(lane >= sh, g, 0)
    return inc - x, inc


def _tab_get(tab16lo, tab16hi, i):
    a = tab16lo.at[jnp.full((16,), jnp.minimum(i, 15), jnp.int32)].get(
        mode="promise_in_bounds")
    b = tab16hi.at[jnp.full((16,), jnp.clip(i - 16, 0, 15), jnp.int32)].get(
        mode="promise_in_bounds")
    return jnp.where(i < 16, a, b)[0]


def _b2_body(srcp, dstp, cnt_hbm, barr, astart_out, swin, dwin, call, posb,
             pbuf, vbuf, atab, offs_sh, sem, ssem):
    cid = lax.axis_index("c")
    sid = lax.axis_index("s")
    tid = cid * 16 + sid
    lane = lax.iota(jnp.int32, 16)

    @pl.when(sid == 0)
    def _():
        pltpu.sync_copy(cnt_hbm, call)
        for i in range(8):
            atab[pl.ds(i * 16, 16)] = jnp.zeros((16,), jnp.int32)

        def _bkt(b, carry):
            def _tile(t, carry):
                idx = t * CPT + b * 16
                c16 = call[pl.ds(idx, 16)]
                excl, inc = _excl_scan16(c16, lane)
                call[pl.ds(idx, 16)] = excl + carry
                tot = inc.at[jnp.full((16,), 15, jnp.int32)].get(
                    mode="promise_in_bounds")
                nxt = carry + tot + 15
                return nxt - (nxt & 15)

            carry = lax.fori_loop(0, 32, _tile, carry)
            plsc.store_scatter(atab, [jnp.full((16,), 64, jnp.int32) + b],
                               carry, mask=lane == 0)
            carry = lax.shift_left(
                lax.shift_right_logical(carry + 511, 9), 9)
            plsc.store_scatter(atab, [jnp.full((16,), 1, jnp.int32) + b],
                               carry, mask=lane == 0)
            return carry

        lax.fori_loop(0, NBK, _bkt, jnp.zeros((16,), jnp.int32))
        pltpu.sync_copy(call, offs_sh.at[pl.ds(0, 32 * CPT)])
        pltpu.sync_copy(atab.at[pl.ds(0, 96)], offs_sh.at[pl.ds(32 * CPT, 96)])

        @pl.when(cid == 0)
        def _():
            pltpu.sync_copy(atab.at[pl.ds(0, 32)], astart_out)

    plsc.subcore_barrier()
    pltpu.sync_copy(offs_sh.at[pl.ds(pl.multiple_of(tid * CPT, 8), CPT)], posb)
    pltpu.sync_copy(offs_sh.at[pl.ds(32 * CPT, 96)], atab.at[pl.ds(0, 96)])

    def _win(w, _):
        base_row = tid * ROWS_PER_GTILE + w * 8
        c1 = pltpu.async_copy(srcp.at[pl.ds(base_row, 8)], swin, sem)
        c2 = pltpu.async_copy(dstp.at[pl.ds(base_row, 8)], dwin, sem)
        c1.wait()
        c2.wait()

        def _srow(u, _):
            for rr in range(4):
                for g in range(8):
                    s16 = swin[u * 4 + rr, pl.ds(g * 16, 16)]
                    d16 = dwin[u * 4 + rr, pl.ds(g * 16, 16)]
                    v = lax.shift_left(s16, PSH) | (d16 & 16383)
                    cidx = lax.shift_left(
                        lax.shift_right_logical(d16, 12), 4) + lane
                    p = plsc.load_gather(posb, [cidx])
                    plsc.store_scatter(posb, [cidx], p + 1)
                    pbuf[rr, pl.ds(g * 16, 16)] = p
                    vbuf[rr, pl.ds(g * 16, 16)] = v
            for j in range(4):
                pltpu.async_copy(vbuf.at[j], barr.at[pbuf.at[j]],
                                 ssem).wait()
            return 0

        lax.fori_loop(0, 2, _srow, 0)
        return 0

    lax.fori_loop(0, ROWS_PER_GTILE // 8, _win, 0)

    padv = lax.shift_left(jnp.int32(N) + lane * 8, PSH) | 8191
    for b in range(NBK):
        myend = posb[pl.ds(b * 16, 16)][15]
        aend = myend + 15
        aend = aend - (aend & 15)
        pos = myend + lane
        pos = jnp.where(pos < aend, pos, DUMP + b * 16 + lane)
        slot = b % 8
        pbuf[0, pl.ds(slot * 16, 16)] = pos
        vbuf[0, pl.ds(slot * 16, 16)] = padv
        if slot == 7 or b == NBK - 1:
            pltpu.async_copy(vbuf.at[0], barr.at[pbuf.at[0]], ssem).wait()
    plsc.subcore_barrier()

    @pl.when(sid < 13)
    def _():
        b = sid * 2 + cid
        alo = atab[pl.ds(0, 16)]
        ahi = atab[pl.ds(16, 16)]
        rlo = atab[pl.ds(64, 16)]
        rhi = atab[pl.ds(80, 16)]
        rend = _tab_get(rlo, rhi, b)
        rend = rend + 15
        rend = rend - (rend & 15)
        nxt = _tab_get(alo, ahi, b + 1)
        padv2 = lax.shift_left(jnp.int32(N) + lane * 8, PSH) | 8191

        def _pk(k, _):
            pos = rend + k * 16 + lane
            m = pos < nxt
            pos = jnp.where(m, pos, DUMP + lane)
            pr = k % 8
            pbuf[0, pl.ds(pr * 16, 16)] = pos
            vbuf[0, pl.ds(pr * 16, 16)] = padv2

            @pl.when(pr == 7)
            def _():
                pltpu.async_copy(vbuf.at[0], barr.at[pbuf.at[0]],
                                 ssem).wait()

            return 0

        lax.fori_loop(0, 32, _pk, 0)


def _bucket2(srcp, dstp, counts):
    mesh = plsc.VectorSubcoreMesh(core_axis_name="c", subcore_axis_name="s")
    return pl.kernel(
        _b2_body,
        out_type=[
            jax.ShapeDtypeStruct((EB,), jnp.int32),
            jax.ShapeDtypeStruct((32,), jnp.int32),
        ],
        mesh=mesh,
        compiler_params=pltpu.CompilerParams(needs_layout_passes=False,
                                             use_tc_tiling_on_sc=False),
        scratch_types=[
            pltpu.VMEM((8, 128), jnp.int32),
            pltpu.VMEM((8, 128), jnp.int32),
            pltpu.VMEM((32 * CPT,), jnp.int32),
            pltpu.VMEM((CPT,), jnp.int32),
            pltpu.VMEM((4, 128), jnp.int32),
            pltpu.VMEM((4, 128), jnp.int32),
            pltpu.VMEM((128,), jnp.int32),
            pltpu.VMEM_SHARED((32 * CPT + 96,), jnp.int32),
            pltpu.SemaphoreType.DMA,
            pltpu.SemaphoreType.DMA,
        ],
    )(srcp, dstp, counts)


def _agg_body(f, nchunk, u_hbm, barr, astart, acc_out, ebuf, gsrc, gdst,
              rows, zbuf, atab, acc_sh, sem, gsem):
    cid = lax.axis_index("c")
    sid = lax.axis_index("s")
    stripe = nchunk // 16
    nchunks = NP // nchunk
    bpc = nchunk // 4096
    lmask = nchunk - 1

    def _z16(i, _):
        r = i // (f // 16)
        o = (i % (f // 16)) * 16
        zbuf[r, pl.ds(o, 16)] = jnp.zeros((16,), jnp.float32)
        return 0

    lax.fori_loop(0, 128 * (f // 16), _z16, 0)
    pltpu.sync_copy(astart, atab.at[pl.ds(0, 32)])
    atab[pl.ds(32, 16)] = jnp.zeros((16,), jnp.int32)

    def _chunk(ci, _):
        chunk = 2 * ci + cid
        base = chunk * nchunk
        for k in range(stripe // 128):
            pltpu.sync_copy(zbuf, acc_sh.at[pl.ds(sid * stripe + k * 128,
                                                  128)])
        plsc.subcore_barrier()

        alo = atab[pl.ds(0, 16)]
        ahi = atab[pl.ds(16, 16)]
        s0 = _tab_get(alo, ahi, chunk * bpc)
        s1 = _tab_get(alo, ahi, chunk * bpc + bpc)
        nblk = lax.shift_right_logical(s1 - s0, 9)
        ntile = lax.shift_right_logical(
            jnp.maximum(nblk - sid + 15, 0), 4)

        def _blk(i, _):
            off = pl.multiple_of(s0 + lax.shift_left(sid + i * 16, 9), 512)
            pltpu.async_copy(barr.at[pl.ds(off, 512)], ebuf, sem).wait()

            def _up(k, _):
                pv = ebuf[pl.ds(k * 16, 16)]
                r = k // 8
                o = (k % 8) * 16
                gsrc[r, pl.ds(o, 16)] = lax.shift_right_logical(pv, PSH)
                gdst[r, pl.ds(o, 16)] = pv & lmask
                return 0

            lax.fori_loop(0, 32, _up, 0)
            cps = [pltpu.async_copy(u_hbm.at[gsrc.at[j]],
                                    rows.at[pl.ds(j * 128, 128)], gsem)
                   for j in range(4)]
            for cp in cps:
                cp.wait()
            for j in range(4):
                pltpu.sync_copy(rows.at[pl.ds(j * 128, 128)],
                                acc_sh.at[gdst.at[j]], add=True)
            return 0

        lax.fori_loop(0, ntile, _blk, 0)
        plsc.subcore_barrier()
        pltpu.sync_copy(acc_sh.at[pl.ds(sid * stripe, stripe)],
                        acc_out.at[pl.ds(base + sid * stripe, stripe)])
        plsc.subcore_barrier()
        return 0

    nch = jnp.where(cid == 0, (nchunks + 1) // 2, nchunks // 2)
    lax.fori_loop(0, nch, _chunk, 0)


def _sc_agg(u, barr, astart, nchunk):
    f = u.shape[1]
    mesh = plsc.VectorSubcoreMesh(core_axis_name="c", subcore_axis_name="s")
    return pl.kernel(
        functools.partial(_agg_body, f, nchunk),
        out_type=jax.ShapeDtypeStruct((NP, f), jnp.float32),
        mesh=mesh,
        compiler_params=pltpu.CompilerParams(needs_layout_passes=False,
                                             use_tc_tiling_on_sc=False),
        scratch_types=[
            pltpu.VMEM((512,), jnp.int32),
            pltpu.VMEM((4, 128), jnp.int32),
            pltpu.VMEM((4, 128), jnp.int32),
            pltpu.VMEM((512, f), jnp.float32),
            pltpu.VMEM((128, f), jnp.float32),
            pltpu.VMEM((48,), jnp.int32),
            pltpu.VMEM_SHARED((nchunk, f), jnp.float32),
            pltpu.SemaphoreType.DMA,
            pltpu.SemaphoreType.DMA,
        ],
    )(u, barr, astart)


def _rowmask(i):
    rows = jax.lax.broadcasted_iota(jnp.int32, (BLK, 1), 0) + i * BLK
    return rows < N


def _tc_a_body(x_ref, w_ref, dinv_ref, out_ref):
    i = pl.program_id(0)
    v = dinv_ref[...] * (x_ref[...] @ w_ref[...])
    out_ref[...] = jnp.where(_rowmask(i), v, 0.0)


def _tc_a(x, w, dinv2d):
    fo = w.shape[1]
    return pl.pallas_call(
        _tc_a_body,
        grid=(NU // BLK,),
        in_specs=[
            pl.BlockSpec((BLK, x.shape[1]), lambda i: (i, 0)),
            pl.BlockSpec(w.shape, lambda i: (0, 0)),
            pl.BlockSpec((BLK, 1), lambda i: (i, 0)),
        ],
        out_specs=pl.BlockSpec((BLK, fo), lambda i: (i, 0)),
        out_shape=jax.ShapeDtypeStruct((NU, fo), jnp.float32),
    )(x, w, dinv2d)


def _tc_b_body(acc_ref, u_ref, dinv_ref, b_ref, w_ref, out_ref):
    i = pl.program_id(0)
    dv = dinv_ref[...]
    h = jnp.maximum(dv * (acc_ref[...] + u_ref[...]) + b_ref[...], 0.0)
    out_ref[...] = jnp.where(_rowmask(i), dv * (h @ w_ref[...]), 0.0)


def _tc_b(acc, u, dinv2d, b, w):
    f, fo = w.shape
    return pl.pallas_call(
        _tc_b_body,
        grid=(NU // BLK,),
        in_specs=[
            pl.BlockSpec((BLK, f), lambda i: (i, 0)),
            pl.BlockSpec((BLK, f), lambda i: (i, 0)),
            pl.BlockSpec((BLK, 1), lambda i: (i, 0)),
            pl.BlockSpec((1, f), lambda i: (0, 0)),
            pl.BlockSpec(w.shape, lambda i: (0, 0)),
        ],
        out_specs=pl.BlockSpec((BLK, fo), lambda i: (i, 0)),
        out_shape=jax.ShapeDtypeStruct((NU, fo), jnp.float32),
    )(acc, u, dinv2d, b.reshape(1, -1), w)


def _tc_c_body(acc_ref, u_ref, dinv_ref, b_ref, wn1_ref, bn1_ref, wn2_ref,
               bn2_ref, wg1_ref, bg1_ref, wg2_ref, bg2_ref, nf_ref, gf_ref,
               gsum):
    i = pl.program_id(0)
    dv = dinv_ref[...]
    h3 = dv * (acc_ref[...] + u_ref[...]) + b_ref[...]
    t = jnp.maximum(h3 @ wn1_ref[...] + bn1_ref[...], 0.0)
    nf_ref[...] = jnp.maximum(t @ wn2_ref[...] + bn2_ref[...], 0.0)
    s = jnp.sum(h3, axis=0, keepdims=True)
    prev = jnp.where(i == 0, jnp.zeros_like(s), gsum[...])
    tot = prev + s
    gsum[...] = tot

    @pl.when(i == (N // BLK) - 1)
    def _():
        g = tot * (1.0 / N)
        gg = jnp.maximum(g @ wg1_ref[...] + bg1_ref[...], 0.0)
        gf_ref[...] = jnp.maximum(gg @ wg2_ref[...] + bg2_ref[...], 0.0)


def _tc_c(acc, u, dinv2d, b3, Wn1, bn1, Wn2, bn2, Wg1, bg1, Wg2, bg2):
    full = lambda a: pl.BlockSpec(a.shape, lambda i: tuple(0 for _ in a.shape))
    row = lambda a: pl.BlockSpec((1, a.shape[0]), lambda i: (0, 0))
    return pl.pallas_call(
        _tc_c_body,
        grid=(N // BLK,),
        in_specs=[
            pl.BlockSpec((BLK, 64), lambda i: (i, 0)),
            pl.BlockSpec((BLK, 64), lambda i: (i, 0)),
            pl.BlockSpec((BLK, 1), lambda i: (i, 0)),
            row(b3), full(Wn1), row(bn1), full(Wn2), row(bn2),
            full(Wg1), row(bg1), full(Wg2), row(bg2),
        ],
        out_specs=[
            pl.BlockSpec((BLK, 64), lambda i: (i, 0)),
            pl.BlockSpec((1, 64), lambda i: (0, 0)),
        ],
        out_shape=[
            jax.ShapeDtypeStruct((N, 64), jnp.float32),
            jax.ShapeDtypeStruct((1, 64), jnp.float32),
        ],
        scratch_shapes=[pltpu.VMEM((1, 64), jnp.float32)],
    )(acc, u, dinv2d, b3.reshape(1, -1), Wn1, bn1.reshape(1, -1), Wn2,
      bn2.reshape(1, -1), Wg1, bg1.reshape(1, -1), Wg2, bg2.reshape(1, -1))


def kernel(x, edge_index, W1, b1, W2, b2, W3, b3, Wn1, bn1, Wn2, bn2, Wg1, bg1, Wg2, bg2):
    src = edge_index[0].astype(jnp.int32)
    dst = edge_index[1].astype(jnp.int32)

    padix = jnp.arange(EPAD, dtype=jnp.int32) % 128
    srcp = jnp.concatenate([src, padix]).reshape(ER, 128)
    dstp = jnp.concatenate([dst, NP - 128 + padix]).reshape(ER, 128)
    xp = jnp.concatenate([x, jnp.zeros((NU - N, x.shape[1]), x.dtype)])

    dinv1 = _deg_dinv(dstp)
    dinv2d = dinv1[:, None]

    counts = _bucket1(dstp)
    barr, astart = _bucket2(srcp, dstp, counts)

    u1 = _tc_a(xp, W1, dinv2d)
    acc1 = _sc_agg(u1, barr, astart, 8192)
    u2 = _tc_b(acc1, u1, dinv2d, b1, W2)
    acc2 = _sc_agg(u2, barr, astart, 4096)
    u3 = _tc_b(acc2, u2, dinv2d, b2, W3)
    acc3 = _sc_agg(u3, barr, astart, 8192)
    nf, gf = _tc_c(acc3, u3, dinv2d, b3, Wn1, bn1, Wn2, bn2, Wg1, bg1,
                   Wg2, bg2)
    return (nf, gf)
